# Initial kernel scaffold; baseline (speedup 1.0000x reference)
import jax
import jax.numpy as jnp
from jax.experimental import pallas as pl


def kernel(events_feature, Wq, Wk, Wv, Wpe, Wsa1, Wsa2):
    raise NotImplementedError("write your pallas kernel here")


# speedup vs baseline: 2.9326x; 2.9326x over previous
import jax
import jax.numpy as jnp
from jax.experimental import pallas as pl
from jax.experimental.pallas import tpu as pltpu


def _dot(a, b, dims, precision=jax.lax.Precision.HIGHEST):
    return jax.lax.dot_general(a, b, dimension_numbers=(dims, ((), ())),
                               precision=precision,
                               preferred_element_type=jnp.float32)


def _fps_attention_kernel(x_ref, wk_ref, wv_ref, wpe_ref, wsa1_ref, wsa2_ref,
                          out_ref, sel_ref):
    B, N, C = x_ref.shape
    M = 64
    J = 3 * B
    JC = J * C

    wk = wk_ref[...]
    wv = wv_ref[...]
    wpe = wpe_ref[...]

    dflt = jax.lax.Precision.DEFAULT
    cols = [x_ref[b] for b in range(B)]
    cols += [_dot(x_ref[b], wk, ((1,), (1,)), dflt) for b in range(B)]
    cols += [_dot(x_ref[b], wv, ((1,), (1,)), dflt) for b in range(B)]
    dataF = jnp.concatenate(cols, axis=1)

    lane_jc = jax.lax.broadcasted_iota(jnp.int32, (JC, J), 0) // C
    col_j = jax.lax.broadcasted_iota(jnp.int32, (JC, J), 1)
    gmask = jnp.where(lane_jc == col_j, 1.0, 0.0).astype(jnp.float32)
    gmaskT = jnp.where(
        jax.lax.broadcasted_iota(jnp.int32, (J, JC), 1) // C
        == jax.lax.broadcasted_iota(jnp.int32, (J, JC), 0),
        1.0, 0.0).astype(jnp.float32)

    sq = dataF * dataF
    xx = _dot(sq, gmask, ((1,), (0,)))

    row_iota = jax.lax.broadcasted_iota(jnp.int32, (N, J), 0)
    BIG = jnp.int32(1 << 30)

    def argmax_cols(d):
        mx = jnp.max(d, axis=0, keepdims=True)
        cand = jnp.where(d == mx, row_iota, BIG)
        return jnp.min(cand, axis=0, keepdims=True)

    bary = jnp.sum(dataF, axis=0, keepdims=True) / float(N)
    cc0 = _dot(bary * bary, gmask, ((1,), (0,)))
    baryD = jnp.broadcast_to(bary, (J, JC)) * gmaskT
    d0 = xx - 2.0 * _dot(dataF, baryD, ((1,), (1,))) + cc0
    idx0 = argmax_cols(d0)

    def body(m, carry):
        dist_min, idx = carry
        onehot = jnp.where(row_iota == idx, 1.0, 0.0).astype(jnp.float32)
        sel_ref[pl.ds(m, 1), :] = idx
        cenX = _dot(onehot, dataF, ((0,), (0,)))
        cenD = cenX * gmaskT
        cc = jnp.sum(onehot * xx, axis=0, keepdims=True)
        d = xx - 2.0 * _dot(dataF, cenD, ((1,), (1,))) + cc
        dist_min = jnp.minimum(dist_min, d)
        return dist_min, argmax_cols(dist_min)

    init = (jnp.full((N, J), 1e10, jnp.float32), idx0)
    jax.lax.fori_loop(0, M, body, init)

    sel = sel_ref[...]
    sel_f = sel.astype(jnp.float32)
    col_iota_n = jax.lax.broadcasted_iota(jnp.int32, (M, N), 1)
    m_iota = jax.lax.broadcasted_iota(jnp.int32, (M, M), 1)

    rows = []
    for j in range(J):
        sj = sel_f[:, j:j + 1]
        sjT = jnp.transpose(sj)
        rank = jnp.sum(jnp.where(sjT < sj, 1.0, 0.0), axis=1, keepdims=True)
        pj = jnp.where(rank.astype(jnp.int32) == m_iota, 1.0, 0.0)
        srt = _dot(pj, sj, ((0,), (0,)))
        oh = jnp.where(srt.astype(jnp.int32) == col_iota_n, 1.0, 0.0)
        rows.append(_dot(oh, dataF[:, j * C:(j + 1) * C], ((1,), (0,))))

    w1 = wsa1_ref[...]
    wsa2 = wsa2_ref[...]
    for b in range(B):
        x_m, k_m, v_m = rows[b], rows[B + b], rows[2 * B + b]
        pem = _dot(x_m, wpe, ((1,), (1,)))
        bb = _dot(k_m + pem, w1, ((1,), (1,)))
        z = -bb
        z = z - jnp.max(z, axis=0, keepdims=True)
        e = jnp.exp(z)
        w = e / jnp.sum(e, axis=0, keepdims=True)
        sx = jnp.sum(x_ref[b], axis=0, keepdims=True)
        spex = _dot(sx, wpe, ((1,), (1,)))
        wsum = _dot(w, v_m - pem, ((0,), (0,)))
        r = spex + float(N) * wsum
        o = _dot(r, wsa2, ((1,), (1,)))
        out_ref[b] = jnp.broadcast_to(o, (N, C))


@jax.jit
def kernel(events_feature, Wq, Wk, Wv, Wpe, Wsa1, Wsa2):
    B, N, C = events_feature.shape
    M = 64
    J = 3 * B
    return pl.pallas_call(
        _fps_attention_kernel,
        out_shape=jax.ShapeDtypeStruct((B, N, C), jnp.float32),
        scratch_shapes=[
            pltpu.VMEM((M, J), jnp.int32),
        ],
    )(events_feature, Wk, Wv, Wpe, Wsa1, Wsa2)


# speedup vs baseline: 9.6212x; 3.2808x over previous
import jax
import jax.numpy as jnp
from jax.experimental import pallas as pl
from jax.experimental.pallas import tpu as pltpu


def _dot(a, b, dims, precision=jax.lax.Precision.HIGHEST):
    return jax.lax.dot_general(a, b, dimension_numbers=(dims, ((), ())),
                               precision=precision,
                               preferred_element_type=jnp.float32)


def _fps_attention_kernel(x_ref, wk_ref, wv_ref, wpe_ref, wsa1_ref, wsa2_ref,
                          out_ref, sel_ref, *d_refs):
    B, N, C = x_ref.shape
    M = 64
    J = 3 * B
    JC = J * C

    wk = wk_ref[...]
    wv = wv_ref[...]
    wpe = wpe_ref[...]

    dflt = jax.lax.Precision.DEFAULT
    cols = [x_ref[b] for b in range(B)]
    cols += [_dot(x_ref[b], wk, ((1,), (1,)), dflt) for b in range(B)]
    cols += [_dot(x_ref[b], wv, ((1,), (1,)), dflt) for b in range(B)]
    dataF = jnp.concatenate(cols, axis=1)

    lane_jc = jax.lax.broadcasted_iota(jnp.int32, (JC, J), 0) // C
    col_j = jax.lax.broadcasted_iota(jnp.int32, (JC, J), 1)
    gmask = jnp.where(lane_jc == col_j, 1.0, 0.0).astype(jnp.float32)

    sq = dataF * dataF
    xx = _dot(sq, gmask, ((1,), (0,)))
    xxT = jnp.transpose(xx)

    for j in range(J):
        dj = dataF[:, j * C:(j + 1) * C]
        g = _dot(dj, dj, ((1,), (1,)))
        d_refs[j][...] = xx[:, j:j + 1] + xxT[j:j + 1, :] - 2.0 * g

    lane_iota = jax.lax.broadcasted_iota(jnp.int32, (J, N), 1)
    BIG = jnp.int32(1 << 30)

    gmaskT = jnp.where(
        jax.lax.broadcasted_iota(jnp.int32, (J, JC), 1) // C
        == jax.lax.broadcasted_iota(jnp.int32, (J, JC), 0),
        1.0, 0.0).astype(jnp.float32)
    bary = jnp.sum(dataF, axis=0, keepdims=True) / float(N)
    cc0 = _dot(bary * bary, gmask, ((1,), (0,)))
    baryD = jnp.broadcast_to(bary, (J, JC)) * gmaskT
    d0 = xx - 2.0 * _dot(dataF, baryD, ((1,), (1,))) + cc0
    d0T = jnp.transpose(d0)

    def argmax_rows(d):
        mx = jnp.max(d, axis=1, keepdims=True)
        cand = jnp.where(d == mx, lane_iota, BIG)
        vec = jnp.min(cand, axis=1, keepdims=True)
        scalars = [jnp.min(cand[j:j + 1, :], axis=1, keepdims=True)[0, 0]
                   for j in range(J)]
        return vec, scalars

    idx0_vec, idx0_s = argmax_rows(d0T)

    def body(m, carry):
        dist_min, idx_vec, idx_s = carry
        sel_ref[pl.ds(m, 1), :] = jnp.transpose(idx_vec)
        rows = jnp.concatenate(
            [d_refs[j][pl.ds(idx_s[j], 1), :] for j in range(J)], axis=0)
        dist_min = jnp.minimum(dist_min, rows)
        vec, scalars = argmax_rows(dist_min)
        return dist_min, vec, tuple(scalars)

    init = (jnp.full((J, N), 1e10, jnp.float32), idx0_vec, tuple(idx0_s))
    jax.lax.fori_loop(0, M, body, init)

    sel = sel_ref[...]
    sel_f = sel.astype(jnp.float32)
    col_iota_n = jax.lax.broadcasted_iota(jnp.int32, (M, N), 1)
    m_iota = jax.lax.broadcasted_iota(jnp.int32, (M, M), 1)

    rows = []
    for j in range(J):
        sj = sel_f[:, j:j + 1]
        sjT = jnp.transpose(sj)
        rank = jnp.sum(jnp.where(sjT < sj, 1.0, 0.0), axis=1, keepdims=True)
        pj = jnp.where(rank.astype(jnp.int32) == m_iota, 1.0, 0.0)
        srt = _dot(pj, sj, ((0,), (0,)))
        oh = jnp.where(srt.astype(jnp.int32) == col_iota_n, 1.0, 0.0)
        rows.append(_dot(oh, dataF[:, j * C:(j + 1) * C], ((1,), (0,))))

    w1 = wsa1_ref[...]
    wsa2 = wsa2_ref[...]
    for b in range(B):
        x_m, k_m, v_m = rows[b], rows[B + b], rows[2 * B + b]
        pem = _dot(x_m, wpe, ((1,), (1,)))
        bb = _dot(k_m + pem, w1, ((1,), (1,)))
        z = -bb
        z = z - jnp.max(z, axis=0, keepdims=True)
        e = jnp.exp(z)
        w = e / jnp.sum(e, axis=0, keepdims=True)
        sx = jnp.sum(x_ref[b], axis=0, keepdims=True)
        spex = _dot(sx, wpe, ((1,), (1,)))
        wsum = _dot(w, v_m - pem, ((0,), (0,)))
        r = spex + float(N) * wsum
        o = _dot(r, wsa2, ((1,), (1,)))
        out_ref[b] = jnp.broadcast_to(o, (N, C))


@jax.jit
def kernel(events_feature, Wq, Wk, Wv, Wpe, Wsa1, Wsa2):
    B, N, C = events_feature.shape
    M = 64
    J = 3 * B
    return pl.pallas_call(
        _fps_attention_kernel,
        out_shape=jax.ShapeDtypeStruct((B, N, C), jnp.float32),
        scratch_shapes=[pltpu.VMEM((M, J), jnp.int32)]
        + [pltpu.VMEM((N, N), jnp.float32) for _ in range(J)],
    )(events_feature, Wk, Wv, Wpe, Wsa1, Wsa2)


# speedup vs baseline: 9.6827x; 1.0064x over previous
import jax
import jax.numpy as jnp
from jax.experimental import pallas as pl
from jax.experimental.pallas import tpu as pltpu


def _dot(a, b, dims, precision=jax.lax.Precision.HIGHEST):
    return jax.lax.dot_general(a, b, dimension_numbers=(dims, ((), ())),
                               precision=precision,
                               preferred_element_type=jnp.float32)


def _fps_attention_kernel(x_ref, wk_ref, wv_ref, wpe_ref, wsa1_ref, wsa2_ref,
                          out_ref, sel_ref, *d_refs):
    B, N, C = x_ref.shape
    M = 64
    J = 3 * B
    JC = J * C

    wk = wk_ref[...]
    wv = wv_ref[...]
    wpe = wpe_ref[...]

    dflt = jax.lax.Precision.DEFAULT
    cols = [x_ref[b] for b in range(B)]
    cols += [_dot(x_ref[b], wk, ((1,), (1,)), dflt) for b in range(B)]
    cols += [_dot(x_ref[b], wv, ((1,), (1,)), dflt) for b in range(B)]
    dataF = jnp.concatenate(cols, axis=1)

    lane_jc = jax.lax.broadcasted_iota(jnp.int32, (JC, J), 0) // C
    col_j = jax.lax.broadcasted_iota(jnp.int32, (JC, J), 1)
    gmask = jnp.where(lane_jc == col_j, 1.0, 0.0).astype(jnp.float32)

    sq = dataF * dataF
    xx = _dot(sq, gmask, ((1,), (0,)))
    xxT = jnp.transpose(xx)

    for j in range(J):
        dj = dataF[:, j * C:(j + 1) * C]
        g = _dot(dj, dj, ((1,), (1,)))
        d_refs[j][...] = xx[:, j:j + 1] + xxT[j:j + 1, :] - 2.0 * g

    lane_iota = jax.lax.broadcasted_iota(jnp.int32, (J, N), 1)
    BIG = jnp.int32(1 << 30)

    gmaskT = jnp.where(
        jax.lax.broadcasted_iota(jnp.int32, (J, JC), 1) // C
        == jax.lax.broadcasted_iota(jnp.int32, (J, JC), 0),
        1.0, 0.0).astype(jnp.float32)
    bary = jnp.sum(dataF, axis=0, keepdims=True) / float(N)
    cc0 = _dot(bary * bary, gmask, ((1,), (0,)))
    baryD = jnp.broadcast_to(bary, (J, JC)) * gmaskT
    d0 = xx - 2.0 * _dot(dataF, baryD, ((1,), (1,))) + cc0
    d0T = jnp.transpose(d0)

    def argmax_rows(d):
        mx = jnp.max(d, axis=1, keepdims=True)
        cand = jnp.where(d == mx, lane_iota, BIG)
        vec = jnp.min(cand, axis=1, keepdims=True)
        scalars = [vec[j, 0] for j in range(J)]
        return vec, scalars

    idx0_vec, idx0_s = argmax_rows(d0T)

    def body(m, carry):
        dist_min, idx_vec, idx_s = carry
        sel_ref[pl.ds(m, 1), :] = jnp.transpose(idx_vec)
        rows = jnp.concatenate(
            [d_refs[j][pl.ds(idx_s[j], 1), :] for j in range(J)], axis=0)
        dist_min = jnp.minimum(dist_min, rows)
        vec, scalars = argmax_rows(dist_min)
        return dist_min, vec, tuple(scalars)

    init = (jnp.full((J, N), 1e10, jnp.float32), idx0_vec, tuple(idx0_s))
    jax.lax.fori_loop(0, M, body, init)

    sel = sel_ref[...]
    sel_f = sel.astype(jnp.float32)
    col_iota_n = jax.lax.broadcasted_iota(jnp.int32, (M, N), 1)
    m_iota = jax.lax.broadcasted_iota(jnp.int32, (M, M), 1)

    rows = []
    for j in range(J):
        sj = sel_f[:, j:j + 1]
        sjT = jnp.transpose(sj)
        rank = jnp.sum(jnp.where(sjT < sj, 1.0, 0.0), axis=1, keepdims=True)
        pj = jnp.where(rank.astype(jnp.int32) == m_iota, 1.0, 0.0)
        srt = _dot(pj, sj, ((0,), (0,)))
        oh = jnp.where(srt.astype(jnp.int32) == col_iota_n, 1.0, 0.0)
        rows.append(_dot(oh, dataF[:, j * C:(j + 1) * C], ((1,), (0,))))

    w1 = wsa1_ref[...]
    wsa2 = wsa2_ref[...]
    for b in range(B):
        x_m, k_m, v_m = rows[b], rows[B + b], rows[2 * B + b]
        pem = _dot(x_m, wpe, ((1,), (1,)))
        bb = _dot(k_m + pem, w1, ((1,), (1,)))
        z = -bb
        z = z - jnp.max(z, axis=0, keepdims=True)
        e = jnp.exp(z)
        w = e / jnp.sum(e, axis=0, keepdims=True)
        sx = jnp.sum(x_ref[b], axis=0, keepdims=True)
        spex = _dot(sx, wpe, ((1,), (1,)))
        wsum = _dot(w, v_m - pem, ((0,), (0,)))
        r = spex + float(N) * wsum
        o = _dot(r, wsa2, ((1,), (1,)))
        out_ref[b] = jnp.broadcast_to(o, (N, C))


@jax.jit
def kernel(events_feature, Wq, Wk, Wv, Wpe, Wsa1, Wsa2):
    B, N, C = events_feature.shape
    M = 64
    J = 3 * B
    return pl.pallas_call(
        _fps_attention_kernel,
        out_shape=jax.ShapeDtypeStruct((B, N, C), jnp.float32),
        scratch_shapes=[pltpu.VMEM((M, J), jnp.int32)]
        + [pltpu.VMEM((N, N), jnp.float32) for _ in range(J)],
    )(events_feature, Wk, Wv, Wpe, Wsa1, Wsa2)
